# baseline (speedup 1.0000x reference)
import functools

import numpy as np
import jax
import jax.numpy as jnp
from jax.experimental import pallas as pl
from jax.experimental.pallas import tpu as pltpu

_RATIO = 0.6
_RATE = 0.5
_FILL_VALUE = 1
_LANES = 128
_SUBLANES = 8


def _make_grid_mask_np(H, W, ratio, rng):
    mask_size = int(max(H, W) * 2)
    lo = int(min(H * 0.5, W * 0.3))
    hi = int(max(H * 0.5, W * 0.3)) + 1
    gridblock = int(rng.integers(lo, hi))
    if ratio == 1:
        length = int(rng.integers(1, gridblock + 1))
    else:
        length = int(min(max(int(gridblock * ratio + 0.5), 1), gridblock - 1))
    mask = np.zeros((mask_size, mask_size), dtype=np.int32)
    for _ in range(2):
        start_w = int(rng.integers(0, gridblock + 1))
        for i in range(mask_size // gridblock):
            start = gridblock * i + start_w
            end = min(start + length, mask_size)
            if end > start:
                mask[start:end, :] = _FILL_VALUE
        mask = mask.T.copy()
    top = (mask_size - H) // 2
    left = (mask_size - W) // 2
    return mask[top:top + H, left:left + W]


@functools.lru_cache(maxsize=None)
def _mask_factors(B, H, W):
    rng = np.random.default_rng(0)
    masks = []
    for _ in range(B):
        m = _make_grid_mask_np(H, W, _RATIO, rng)
        rate_cond = rng.random() < _RATE
        if not rate_cond:
            m = np.ones((H, W), dtype=np.int32)
        masks.append(m)
    masks = np.stack(masks).astype(np.float32)
    row = masks.min(axis=2)
    col = masks.min(axis=1)
    rec = np.maximum(row[:, :, None], col[:, None, :])
    assert np.array_equal(rec, masks), "mask not row/col separable"
    rowb = np.repeat(row[:, :, None], _LANES, axis=2).astype(np.int8)
    colb = np.repeat(col[:, None, :], _SUBLANES, axis=1).astype(np.int8)
    ones = np.all(masks == 1, axis=(1, 2)).astype(np.int32)
    return rowb, colb, ones


def _body(ones_ref, row_ref, col_ref, img_ref, out_ref, m_ref):
    H = row_ref.shape[1]
    W = col_ref.shape[2]
    b = pl.program_id(0)
    c = pl.program_id(1)
    is_ones = ones_ref[b] != 0

    @pl.when(jnp.logical_not(is_ones) & (c == 0))
    def _build():
        rowb = row_ref[0]
        colb = col_ref[0]
        row_full = jnp.concatenate([rowb] * (W // _LANES), axis=1)
        col_full = jnp.concatenate([colb] * (H // _SUBLANES), axis=0)
        m_ref[...] = (row_full | col_full).astype(jnp.float32)

    @pl.when(is_ones)
    def _copy():
        out_ref[0, 0] = img_ref[0, 0]

    @pl.when(jnp.logical_not(is_ones))
    def _masked():
        out_ref[0, 0] = img_ref[0, 0] * m_ref[...]


def kernel(images):
    B, H, W, C = images.shape
    rowb, colb, ones = _mask_factors(B, H, W)
    rowb = jnp.asarray(rowb)
    colb = jnp.asarray(colb)
    ones = jnp.asarray(ones)
    img_t = jnp.transpose(images, (0, 3, 1, 2))

    out = pl.pallas_call(
        _body,
        grid_spec=pltpu.PrefetchScalarGridSpec(
            num_scalar_prefetch=1,
            grid=(B, C),
            in_specs=[
                pl.BlockSpec((1, H, _LANES), lambda b, c, ones_ref: (b, 0, 0)),
                pl.BlockSpec((1, _SUBLANES, W), lambda b, c, ones_ref: (b, 0, 0)),
                pl.BlockSpec((1, 1, H, W), lambda b, c, ones_ref: (b, c, 0, 0)),
            ],
            out_specs=pl.BlockSpec((1, 1, H, W), lambda b, c, ones_ref: (b, c, 0, 0)),
            scratch_shapes=[pltpu.VMEM((H, W), jnp.float32)],
        ),
        out_shape=jax.ShapeDtypeStruct((B, C, H, W), jnp.float32),
        compiler_params=pltpu.CompilerParams(
            dimension_semantics=("arbitrary", "arbitrary"),
        ),
    )(ones, rowb, colb, img_t)
    return jnp.transpose(out, (0, 2, 3, 1))


# speedup vs baseline: 1.5737x; 1.5737x over previous
import functools

import numpy as np
import jax
import jax.numpy as jnp
from jax.experimental import pallas as pl
from jax.experimental.pallas import tpu as pltpu

_RATIO = 0.6
_RATE = 0.5
_FILL_VALUE = 1
_LANES = 128
_SUBLANES = 8


def _make_grid_mask_np(H, W, ratio, rng):
    mask_size = int(max(H, W) * 2)
    lo = int(min(H * 0.5, W * 0.3))
    hi = int(max(H * 0.5, W * 0.3)) + 1
    gridblock = int(rng.integers(lo, hi))
    if ratio == 1:
        length = int(rng.integers(1, gridblock + 1))
    else:
        length = int(min(max(int(gridblock * ratio + 0.5), 1), gridblock - 1))
    mask = np.zeros((mask_size, mask_size), dtype=np.int32)
    for _ in range(2):
        start_w = int(rng.integers(0, gridblock + 1))
        for i in range(mask_size // gridblock):
            start = gridblock * i + start_w
            end = min(start + length, mask_size)
            if end > start:
                mask[start:end, :] = _FILL_VALUE
        mask = mask.T.copy()
    top = (mask_size - H) // 2
    left = (mask_size - W) // 2
    return mask[top:top + H, left:left + W]


@functools.lru_cache(maxsize=None)
def _mask_factors(B, H, W):
    rng = np.random.default_rng(0)
    masks = []
    for _ in range(B):
        m = _make_grid_mask_np(H, W, _RATIO, rng)
        rate_cond = rng.random() < _RATE
        if not rate_cond:
            m = np.ones((H, W), dtype=np.int32)
        masks.append(m)
    masks = np.stack(masks).astype(np.float32)
    row = masks.min(axis=2)
    col = masks.min(axis=1)
    rec = np.maximum(row[:, :, None], col[:, None, :])
    assert np.array_equal(rec, masks), "mask not row/col separable"
    rowb = np.repeat(row[:, :, None], _LANES, axis=2).astype(np.int8)
    colb = np.repeat(col[:, None, :], _SUBLANES, axis=1).astype(np.int8)
    return rowb, colb


def _body(row_ref, col_ref, img_ref, out_ref):
    BBLK = img_ref.shape[0]
    C = img_ref.shape[1]
    H = row_ref.shape[1]
    W = col_ref.shape[2]
    for i in range(BBLK):
        rowb = row_ref[i]
        colb = col_ref[i]
        row_full = jnp.concatenate([rowb] * (W // _LANES), axis=1)
        col_full = jnp.concatenate([colb] * (H // _SUBLANES), axis=0)
        m = (row_full | col_full).astype(jnp.float32)
        for c in range(C):
            out_ref[i, c] = img_ref[i, c] * m


def kernel(images):
    B, H, W, C = images.shape
    rowb, colb = _mask_factors(B, H, W)
    rowb = jnp.asarray(rowb)
    colb = jnp.asarray(colb)
    img_t = jnp.transpose(images, (0, 3, 1, 2))

    BBLK = 2
    grid = (B // BBLK,)
    out = pl.pallas_call(
        _body,
        grid=grid,
        in_specs=[
            pl.BlockSpec((BBLK, H, _LANES), lambda b: (b, 0, 0)),
            pl.BlockSpec((BBLK, _SUBLANES, W), lambda b: (b, 0, 0)),
            pl.BlockSpec((BBLK, C, H, W), lambda b: (b, 0, 0, 0)),
        ],
        out_specs=pl.BlockSpec((BBLK, C, H, W), lambda b: (b, 0, 0, 0)),
        out_shape=jax.ShapeDtypeStruct((B, C, H, W), jnp.float32),
        compiler_params=pltpu.CompilerParams(
            dimension_semantics=("arbitrary",),
        ),
    )(rowb, colb, img_t)
    return jnp.transpose(out, (0, 2, 3, 1))


# speedup vs baseline: 1.5847x; 1.0070x over previous
import functools

import numpy as np
import jax
import jax.numpy as jnp
from jax.experimental import pallas as pl
from jax.experimental.pallas import tpu as pltpu

_RATIO = 0.6
_RATE = 0.5
_FILL_VALUE = 1
_LANES = 128
_SUBLANES = 8


def _make_grid_mask_np(H, W, ratio, rng):
    mask_size = int(max(H, W) * 2)
    lo = int(min(H * 0.5, W * 0.3))
    hi = int(max(H * 0.5, W * 0.3)) + 1
    gridblock = int(rng.integers(lo, hi))
    if ratio == 1:
        length = int(rng.integers(1, gridblock + 1))
    else:
        length = int(min(max(int(gridblock * ratio + 0.5), 1), gridblock - 1))
    mask = np.zeros((mask_size, mask_size), dtype=np.int32)
    for _ in range(2):
        start_w = int(rng.integers(0, gridblock + 1))
        for i in range(mask_size // gridblock):
            start = gridblock * i + start_w
            end = min(start + length, mask_size)
            if end > start:
                mask[start:end, :] = _FILL_VALUE
        mask = mask.T.copy()
    top = (mask_size - H) // 2
    left = (mask_size - W) // 2
    return mask[top:top + H, left:left + W]


@functools.lru_cache(maxsize=None)
def _mask_factors(B, H, W):
    rng = np.random.default_rng(0)
    masks = []
    for _ in range(B):
        m = _make_grid_mask_np(H, W, _RATIO, rng)
        rate_cond = rng.random() < _RATE
        if not rate_cond:
            m = np.ones((H, W), dtype=np.int32)
        masks.append(m)
    masks = np.stack(masks).astype(np.float32)
    row = masks.min(axis=2)
    col = masks.min(axis=1)
    rec = np.maximum(row[:, :, None], col[:, None, :])
    assert np.array_equal(rec, masks), "mask not row/col separable"
    rowb = np.repeat(row[:, :, None], _LANES, axis=2).astype(np.int8)
    colb = np.repeat(col[:, None, :], _SUBLANES, axis=1).astype(np.int8)
    return rowb, colb


def _body(row_ref, col_ref, img_ref, out_ref):
    BBLK = img_ref.shape[0]
    C = img_ref.shape[1]
    H = row_ref.shape[1]
    W = col_ref.shape[2]
    for i in range(BBLK):
        rowb = row_ref[i]
        colb = col_ref[i]
        row_full = jnp.concatenate([rowb] * (W // _LANES), axis=1)
        col_full = jnp.concatenate([colb] * (H // _SUBLANES), axis=0)
        m = (row_full | col_full).astype(jnp.float32)
        for c in range(C):
            out_ref[i, c] = img_ref[i, c] * m


def kernel(images):
    B, H, W, C = images.shape
    rowb, colb = _mask_factors(B, H, W)
    rowb = jnp.asarray(rowb)
    colb = jnp.asarray(colb)
    img_t = jnp.transpose(images, (0, 3, 1, 2))

    BBLK = 4
    grid = (B // BBLK,)
    out = pl.pallas_call(
        _body,
        grid=grid,
        in_specs=[
            pl.BlockSpec((BBLK, H, _LANES), lambda b: (b, 0, 0)),
            pl.BlockSpec((BBLK, _SUBLANES, W), lambda b: (b, 0, 0)),
            pl.BlockSpec((BBLK, C, H, W), lambda b: (b, 0, 0, 0)),
        ],
        out_specs=pl.BlockSpec((BBLK, C, H, W), lambda b: (b, 0, 0, 0)),
        out_shape=jax.ShapeDtypeStruct((B, C, H, W), jnp.float32),
        compiler_params=pltpu.CompilerParams(
            dimension_semantics=("arbitrary",),
        ),
    )(rowb, colb, img_t)
    return jnp.transpose(out, (0, 2, 3, 1))
